# baseline (speedup 1.0000x reference)
import functools

import jax
import jax.numpy as jnp
from jax import lax
from jax.experimental import pallas as pl
from jax.experimental.pallas import tpu as pltpu
from jax.experimental.pallas import tpu_sc as plsc

_B = 4096
_L = 50
_LP = 56
_D = 128
_H1 = 256
_H2 = 128
_NCLS = 50
_EPS = 1e-5

_NC = 2
_NS = 16
_NW = _NC * _NS
_EPW = _B // _NW
_PAIRS = _EPW // 2
_PW = 2 * _LP
_LANES = 16
_VPR = _D // _LANES


def _sc_pool_body(tok_hbm, emb_hbm, out_hbm, tok_v, rows, pooled_v, sem0, sem1):
    c = lax.axis_index("c")
    s = lax.axis_index("s")
    w = c * _NS + s

    pltpu.sync_copy(tok_hbm.at[pl.ds(w * _PAIRS, _PAIRS)], tok_v)

    sems = (sem0, sem1)

    def fire(i, b):
        pltpu.make_async_copy(
            emb_hbm.at[tok_v.at[i]], rows.at[b], sems[b]
        ).start()

    def wait(b):
        pltpu.make_async_copy(
            emb_hbm.at[pl.ds(0, _PW)], rows.at[b], sems[b]
        ).wait()

    fire(0, 0)

    def pair_step(k, carry):
        i0 = 2 * k
        for b in (0, 1):
            i = i0 + b
            wait(b)

            @pl.when(i + 1 < _PAIRS)
            def _():
                fire(i + 1, 1 - b)

            buf = rows.at[b]
            for e in (0, 1):
                base = e * _LP

                def acc_body(r, acc, buf=buf, base=base):
                    return tuple(
                        acc[v] + buf[base + r, pl.ds(v * _LANES, _LANES)]
                        for v in range(_VPR)
                    )

                acc = lax.fori_loop(
                    0, _LP, acc_body,
                    tuple(jnp.zeros((_LANES,), jnp.float32)
                          for _ in range(_VPR)),
                )
                for v in range(_VPR):
                    pooled_v[2 * i + e, pl.ds(v * _LANES, _LANES)] = acc[v]
        return carry

    lax.fori_loop(0, _PAIRS // 2, pair_step, 0)

    pltpu.sync_copy(pooled_v, out_hbm.at[pl.ds(w * _EPW, _EPW)])


_sc_pool = functools.partial(
    pl.kernel,
    out_type=jax.ShapeDtypeStruct((_B, _D), jnp.float32),
    mesh=plsc.VectorSubcoreMesh(core_axis_name="c", subcore_axis_name="s"),
    scratch_types=[
        pltpu.VMEM((_NW * _PAIRS // _NW, _PW), jnp.int32),
        pltpu.VMEM((2, _PW, _D), jnp.float32),
        pltpu.VMEM((_EPW, _D), jnp.float32),
        pltpu.SemaphoreType.DMA,
        pltpu.SemaphoreType.DMA,
    ],
)(_sc_pool_body)


def _mlp_body(ps, w1, b1, g1, be1, w2, b2, g2, be2, wout, bout, out):
    x = ps[...] * (1.0 / _L)

    h = lax.dot_general(x, w1[...], (((1,), (1,)), ((), ())),
                        preferred_element_type=jnp.float32) + b1[...]
    mu = jnp.mean(h, axis=0, keepdims=True)
    d = h - mu
    var = jnp.mean(d * d, axis=0, keepdims=True)
    h = g1[...] * d / jnp.sqrt(var + _EPS) + be1[...]
    h = jnp.maximum(h, 0.0)

    h = lax.dot_general(h, w2[...], (((1,), (1,)), ((), ())),
                        preferred_element_type=jnp.float32) + b2[...]
    mu = jnp.mean(h, axis=0, keepdims=True)
    d = h - mu
    var = jnp.mean(d * d, axis=0, keepdims=True)
    h = g2[...] * d / jnp.sqrt(var + _EPS) + be2[...]
    h = jnp.maximum(h, 0.0)

    out[...] = lax.dot_general(h, wout[...], (((1,), (1,)), ((), ())),
                               preferred_element_type=jnp.float32) + bout[...]


_mlp = pl.pallas_call(
    _mlp_body,
    out_shape=jax.ShapeDtypeStruct((_B, _NCLS), jnp.float32),
)


def kernel(token_ids, emb, W1, b1, g1, be1, W2, b2, g2, be2, Wout, bout):
    tok = token_ids.astype(jnp.int32)
    tokp = jnp.pad(tok, ((0, 0), (0, _LP - _L)))
    tokp = tokp.reshape(_B // 2, _PW)
    pooled_sum = _sc_pool(tokp, emb)
    return _mlp(
        pooled_sum,
        W1, b1.reshape(1, -1), g1.reshape(1, -1), be1.reshape(1, -1),
        W2, b2.reshape(1, -1), g2.reshape(1, -1), be2.reshape(1, -1),
        Wout, bout.reshape(1, -1),
    )
